# initial kernel scaffold (unmeasured)
import jax
import jax.numpy as jnp
from jax import lax
from jax.experimental import pallas as pl
from jax.experimental.pallas import tpu as pltpu

N_DEV = 16
B, H, D, BS, NPAGES_LOCAL = 8, 8, 64, 16, 64
T_LOCAL = NPAGES_LOCAL * BS


def kernel(Q, K, V, bt, lens):
    lens2 = lens.reshape(B, 1)

    def body(q_ref, k_ref, v_ref, bt_ref, lens_ref, out_ref,
             gather_ref, send_sems, recv_sems):
        my_pos = lax.axis_index("i")
        left = lax.rem(my_pos - 1 + N_DEV, N_DEV)
        right = lax.rem(my_pos + 1, N_DEV)

        g_row = (lax.broadcasted_iota(jnp.int32, (1, T_LOCAL), 1) // BS
                 + my_pos * NPAGES_LOCAL)
        lens_v = lens_ref[:, :]

        def cnt_step(j, cnt):
            pj = bt_ref[:, pl.ds(j, 1)]
            hit = (pj == g_row) & (j < lens_v)
            return cnt + hit.astype(jnp.float32)

        cnt = lax.fori_loop(0, 64, cnt_step,
                            jnp.zeros((B, T_LOCAL), jnp.float32))
        valid = cnt > 0.0

        scale = D ** -0.5
        for h in range(H):
            q_h = q_ref[:, 0, h, :].astype(jnp.bfloat16)
            k_h = k_ref[:, :, h, :].reshape(T_LOCAL, D).astype(jnp.bfloat16)
            s = lax.dot_general(
                q_h, k_h, (((1,), (1,)), ((), ())),
                preferred_element_type=jnp.float32) * scale
            s = jnp.where(valid, s, -1e30)
            m_h = jnp.max(s, axis=1, keepdims=True)
            p = cnt * jnp.exp(s - m_h)
            l_h = jnp.sum(p, axis=1, keepdims=True)
            v_h = v_ref[:, :, h, :].reshape(T_LOCAL, D).astype(jnp.bfloat16)
            acc_h = lax.dot_general(
                p.astype(jnp.bfloat16), v_h, (((1,), (0,)), ((), ())),
                preferred_element_type=jnp.float32)
            r0 = h * B
            gather_ref[0, r0:r0 + B, 0:D] = acc_h
            gather_ref[0, r0:r0 + B, D:D + 1] = m_h
            gather_ref[0, r0:r0 + B, D + 1:D + 2] = l_h

        barrier_sem = pltpu.get_barrier_semaphore()
        pl.semaphore_signal(barrier_sem, inc=1, device_id=(left,),
                            device_id_type=pl.DeviceIdType.MESH)
        pl.semaphore_signal(barrier_sem, inc=1, device_id=(right,),
                            device_id_type=pl.DeviceIdType.MESH)
        pl.semaphore_wait(barrier_sem, 2)

        for hop in range(N_DEV - 1):
            rdma = pltpu.make_async_remote_copy(
                src_ref=gather_ref.at[hop],
                dst_ref=gather_ref.at[hop + 1],
                send_sem=send_sems.at[hop],
                recv_sem=recv_sems.at[hop],
                device_id=(right,),
                device_id_type=pl.DeviceIdType.MESH,
            )
            rdma.start()
            rdma.wait()

        for h in range(H):
            r0 = h * B
            blk = gather_ref[:, r0:r0 + B, :]
            acc = blk[:, :, 0:D]
            m = blk[:, :, D:D + 1]
            l = blk[:, :, D + 1:D + 2]
            m_star = jnp.max(m, axis=0)
            w = jnp.exp(m - m_star[None])
            l_tot = jnp.sum(l * w, axis=0)
            out_h = jnp.sum(acc * w, axis=0) / l_tot
            out_ref[:, 0, h, :] = out_h

    return pl.pallas_call(
        body,
        out_shape=jax.ShapeDtypeStruct((B, 1, H, D), jnp.float32),
        in_specs=[pl.BlockSpec(memory_space=pltpu.VMEM)] * 5,
        out_specs=pl.BlockSpec(memory_space=pltpu.VMEM),
        scratch_shapes=[
            pltpu.VMEM((N_DEV, B * H, 128), jnp.float32),
            pltpu.SemaphoreType.DMA((N_DEV - 1,)),
            pltpu.SemaphoreType.DMA((N_DEV - 1,)),
        ],
        compiler_params=pltpu.CompilerParams(collective_id=0),
    )(Q, K, V, bt, lens2)


# baseline (device time: 55242 ns/iter reference)
import jax
import jax.numpy as jnp
from jax import lax
from jax.experimental import pallas as pl
from jax.experimental.pallas import tpu as pltpu

N_DEV = 16
B, H, D, BS, NPAGES_LOCAL = 8, 8, 64, 16, 64
T_LOCAL = NPAGES_LOCAL * BS


def kernel(Q, K, V, bt, lens):
    lens2 = lens.reshape(B, 1)

    def body(q_ref, k_ref, v_ref, bt_ref, lens_ref, out_ref,
             gather_ref, send_sems, recv_sems):
        my_pos = lax.axis_index("i")
        left = lax.rem(my_pos - 1 + N_DEV, N_DEV)
        right = lax.rem(my_pos + 1, N_DEV)

        lens_v = lens_ref[:, :]
        jj = lax.broadcasted_iota(jnp.int32, (B, 64), 1)
        btm = jnp.where(jj < lens_v, bt_ref[:, :], -1)
        g3 = (lax.broadcasted_iota(jnp.int32, (1, 1, T_LOCAL), 2) // BS
              + my_pos * NPAGES_LOCAL)
        hit = (btm.reshape(B, 64, 1) == g3)
        cnt = jnp.sum(hit.astype(jnp.float32), axis=1)
        valid = cnt > 0.0

        scale = D ** -0.5
        for h in range(H):
            q_h = q_ref[:, 0, h, :].astype(jnp.bfloat16)
            k_h = k_ref[:, :, h, :].reshape(T_LOCAL, D).astype(jnp.bfloat16)
            s = lax.dot_general(
                q_h, k_h, (((1,), (1,)), ((), ())),
                preferred_element_type=jnp.float32) * scale
            s = jnp.where(valid, s, -1e30)
            m_h = jnp.max(s, axis=1, keepdims=True)
            p = cnt * jnp.exp(s - m_h)
            l_h = jnp.sum(p, axis=1, keepdims=True)
            v_h = v_ref[:, :, h, :].reshape(T_LOCAL, D).astype(jnp.bfloat16)
            acc_h = lax.dot_general(
                p.astype(jnp.bfloat16), v_h, (((1,), (0,)), ((), ())),
                preferred_element_type=jnp.float32)
            r0 = h * B
            gather_ref[0, r0:r0 + B, 0:D] = acc_h
            gather_ref[0, r0:r0 + B, D:D + 1] = m_h
            gather_ref[0, r0:r0 + B, D + 1:D + 2] = l_h

        barrier_sem = pltpu.get_barrier_semaphore()
        pl.semaphore_signal(barrier_sem, inc=1, device_id=(left,),
                            device_id_type=pl.DeviceIdType.MESH)
        pl.semaphore_signal(barrier_sem, inc=1, device_id=(right,),
                            device_id_type=pl.DeviceIdType.MESH)
        pl.semaphore_wait(barrier_sem, 2)

        for hop in range(N_DEV - 1):
            rdma = pltpu.make_async_remote_copy(
                src_ref=gather_ref.at[hop],
                dst_ref=gather_ref.at[hop + 1],
                send_sem=send_sems.at[hop],
                recv_sem=recv_sems.at[hop],
                device_id=(right,),
                device_id_type=pl.DeviceIdType.MESH,
            )
            rdma.start()
            rdma.wait()

        for h in range(H):
            r0 = h * B
            blk = gather_ref[:, r0:r0 + B, :]
            acc = blk[:, :, 0:D]
            m = blk[:, :, D:D + 1]
            l = blk[:, :, D + 1:D + 2]
            m_star = jnp.max(m, axis=0)
            w = jnp.exp(m - m_star[None])
            l_tot = jnp.sum(l * w, axis=0)
            out_h = jnp.sum(acc * w, axis=0) / l_tot
            out_ref[:, 0, h, :] = out_h

    return pl.pallas_call(
        body,
        out_shape=jax.ShapeDtypeStruct((B, 1, H, D), jnp.float32),
        in_specs=[pl.BlockSpec(memory_space=pltpu.VMEM)] * 5,
        out_specs=pl.BlockSpec(memory_space=pltpu.VMEM),
        scratch_shapes=[
            pltpu.VMEM((N_DEV, B * H, 128), jnp.float32),
            pltpu.SemaphoreType.DMA((N_DEV - 1,)),
            pltpu.SemaphoreType.DMA((N_DEV - 1,)),
        ],
        compiler_params=pltpu.CompilerParams(collective_id=0),
    )(Q, K, V, bt, lens2)


# device time: 39518 ns/iter; 1.3979x vs baseline; 1.3979x over previous
import jax
import jax.numpy as jnp
from jax import lax
from jax.experimental import pallas as pl
from jax.experimental.pallas import tpu as pltpu

N_DEV = 16
B, H, D, BS, NPAGES_LOCAL = 8, 8, 64, 16, 64
T_LOCAL = NPAGES_LOCAL * BS


def kernel(Q, K, V, bt, lens):
    lens2 = lens.reshape(B, 1)

    def body(q_ref, k_ref, v_ref, bt_ref, lens_ref, out_ref,
             gather_ref, send_sems, recv_sems):
        my_pos = lax.axis_index("i")

        lens_v = lens_ref[:, :]
        jj = lax.broadcasted_iota(jnp.int32, (B, 64), 1)
        btm = jnp.where(jj < lens_v, bt_ref[:, :], -1)
        g3 = (lax.broadcasted_iota(jnp.int32, (1, 1, T_LOCAL), 2) // BS
              + my_pos * NPAGES_LOCAL)
        hit = (btm.reshape(B, 64, 1) == g3)
        cnt = jnp.sum(hit.astype(jnp.float32), axis=1)
        valid = cnt > 0.0

        scale = D ** -0.5
        for h in range(H):
            q_h = q_ref[:, 0, h, :].astype(jnp.bfloat16)
            k_h = k_ref[:, :, h, :].reshape(T_LOCAL, D).astype(jnp.bfloat16)
            s = lax.dot_general(
                q_h, k_h, (((1,), (1,)), ((), ())),
                preferred_element_type=jnp.float32) * scale
            s = jnp.where(valid, s, -1e30)
            m_h = jnp.max(s, axis=1, keepdims=True)
            p = cnt * jnp.exp(s - m_h)
            l_h = jnp.sum(p, axis=1, keepdims=True)
            v_h = v_ref[:, :, h, :].reshape(T_LOCAL, D).astype(jnp.bfloat16)
            acc_h = lax.dot_general(
                p.astype(jnp.bfloat16), v_h, (((1,), (0,)), ((), ())),
                preferred_element_type=jnp.float32)
            r0 = h * B
            gather_ref[my_pos, r0:r0 + B, 0:D] = acc_h
            gather_ref[my_pos, r0:r0 + B, D:D + 1] = m_h
            gather_ref[my_pos, r0:r0 + B, D + 1:D + 2] = l_h

        barrier_sem = pltpu.get_barrier_semaphore()
        for k in range(4):
            partner = jnp.bitwise_xor(my_pos, 1 << k)
            pl.semaphore_signal(barrier_sem, inc=1, device_id=(partner,),
                                device_id_type=pl.DeviceIdType.MESH)
        pl.semaphore_wait(barrier_sem, 4)

        for k in range(4):
            size = 1 << k
            partner = jnp.bitwise_xor(my_pos, size)
            blk = jnp.bitwise_and(my_pos, (N_DEV - 1) & ~(size - 1))
            rdma = pltpu.make_async_remote_copy(
                src_ref=gather_ref.at[pl.ds(blk, size)],
                dst_ref=gather_ref.at[pl.ds(blk, size)],
                send_sem=send_sems.at[k],
                recv_sem=recv_sems.at[k],
                device_id=(partner,),
                device_id_type=pl.DeviceIdType.MESH,
            )
            rdma.start()
            rdma.wait()

        for h in range(H):
            r0 = h * B
            blk = gather_ref[:, r0:r0 + B, :]
            acc = blk[:, :, 0:D]
            m = blk[:, :, D:D + 1]
            l = blk[:, :, D + 1:D + 2]
            m_star = jnp.max(m, axis=0)
            w = jnp.exp(m - m_star[None])
            l_tot = jnp.sum(l * w, axis=0)
            out_h = jnp.sum(acc * w, axis=0) / l_tot
            out_ref[:, 0, h, :] = out_h

    return pl.pallas_call(
        body,
        out_shape=jax.ShapeDtypeStruct((B, 1, H, D), jnp.float32),
        in_specs=[pl.BlockSpec(memory_space=pltpu.VMEM)] * 5,
        out_specs=pl.BlockSpec(memory_space=pltpu.VMEM),
        scratch_shapes=[
            pltpu.VMEM((N_DEV, B * H, 128), jnp.float32),
            pltpu.SemaphoreType.DMA((4,)),
            pltpu.SemaphoreType.DMA((4,)),
        ],
        compiler_params=pltpu.CompilerParams(collective_id=0),
    )(Q, K, V, bt, lens2)


# device time: 29787 ns/iter; 1.8546x vs baseline; 1.3267x over previous
import jax
import jax.numpy as jnp
from jax import lax
from jax.experimental import pallas as pl
from jax.experimental.pallas import tpu as pltpu

N_DEV = 16
B, H, D, BS, NPAGES_LOCAL = 8, 8, 64, 16, 64
T_LOCAL = NPAGES_LOCAL * BS
HD = H * D
R = H * B


def kernel(Q, K, V, bt, lens):
    Q2 = Q.reshape(B, HD)
    K2 = K.reshape(T_LOCAL, HD)
    V2 = V.reshape(T_LOCAL, HD)
    lens2 = lens.reshape(B, 1)

    def body(q_ref, k_ref, v_ref, bt_ref, lens_ref, out_ref,
             gather_ref, send_sems, recv_sems):
        my_pos = lax.axis_index("i")

        lens_v = lens_ref[:, :]
        jj = lax.broadcasted_iota(jnp.int32, (B, 64), 1)
        btm = jnp.where(jj < lens_v, bt_ref[:, :], -1)
        g3 = (lax.broadcasted_iota(jnp.int32, (1, 1, T_LOCAL), 2) // BS
              + my_pos * NPAGES_LOCAL)
        hit = (btm.reshape(B, 64, 1) == g3)
        cnt = jnp.sum(hit.astype(jnp.float32), axis=1)

        hh = lax.broadcasted_iota(jnp.int32, (H, 1, HD), 0)
        cc = lax.broadcasted_iota(jnp.int32, (H, 1, HD), 2) // D
        qmask = (hh == cc).astype(jnp.bfloat16)
        q_b = q_ref[:, :].astype(jnp.bfloat16)
        qblk = (q_b[None] * qmask).reshape(R, HD)

        k_b = k_ref[:, :].astype(jnp.bfloat16)
        s = lax.dot_general(qblk, k_b, (((1,), (1,)), ((), ())),
                            preferred_element_type=jnp.float32) * (D ** -0.5)
        cnt_r = jnp.broadcast_to(cnt[None], (H, B, T_LOCAL)).reshape(R, T_LOCAL)
        s = jnp.where(cnt_r > 0.0, s, -1e30)
        m = jnp.max(s, axis=1, keepdims=True)
        p = cnt_r * jnp.exp(s - m)
        l = jnp.sum(p, axis=1, keepdims=True)

        v_b = v_ref[:, :].astype(jnp.bfloat16)
        full = lax.dot_general(p.astype(jnp.bfloat16), v_b,
                               (((1,), (0,)), ((), ())),
                               preferred_element_type=jnp.float32)
        rh = lax.broadcasted_iota(jnp.int32, (R, HD), 0) // B
        ch = lax.broadcasted_iota(jnp.int32, (R, HD), 1) // D
        masked = jnp.where(rh == ch, full, 0.0).astype(jnp.bfloat16)
        fold = (lax.broadcasted_iota(jnp.int32, (HD, D), 0) % D
                == lax.broadcasted_iota(jnp.int32, (HD, D), 1)
                ).astype(jnp.bfloat16)
        acc = lax.dot_general(masked, fold, (((1,), (0,)), ((), ())),
                              preferred_element_type=jnp.float32)

        gather_ref[my_pos, :, 0:D] = acc
        gather_ref[my_pos, :, D:D + 1] = m
        gather_ref[my_pos, :, D + 1:D + 2] = l

        barrier_sem = pltpu.get_barrier_semaphore()
        for k in range(4):
            partner = jnp.bitwise_xor(my_pos, 1 << k)
            pl.semaphore_signal(barrier_sem, inc=1, device_id=(partner,),
                                device_id_type=pl.DeviceIdType.MESH)
        pl.semaphore_wait(barrier_sem, 4)

        for k in range(4):
            size = 1 << k
            partner = jnp.bitwise_xor(my_pos, size)
            blk = jnp.bitwise_and(my_pos, (N_DEV - 1) & ~(size - 1))
            rdma = pltpu.make_async_remote_copy(
                src_ref=gather_ref.at[pl.ds(blk, size)],
                dst_ref=gather_ref.at[pl.ds(blk, size)],
                send_sem=send_sems.at[k],
                recv_sem=recv_sems.at[k],
                device_id=(partner,),
                device_id_type=pl.DeviceIdType.MESH,
            )
            rdma.start()
            rdma.wait()

        all_blk = gather_ref[:, :, :]
        acc_a = all_blk[:, :, 0:D]
        m_a = all_blk[:, :, D:D + 1]
        l_a = all_blk[:, :, D + 1:D + 2]
        m_star = jnp.max(m_a, axis=0)
        w = jnp.exp(m_a - m_star[None])
        l_tot = jnp.sum(l_a * w, axis=0)
        out2d = jnp.sum(acc_a * w, axis=0) / l_tot
        for h in range(H):
            out_ref[:, 0, h, :] = out2d[h * B:(h + 1) * B, :]

    return pl.pallas_call(
        body,
        out_shape=jax.ShapeDtypeStruct((B, 1, H, D), jnp.float32),
        in_specs=[pl.BlockSpec(memory_space=pltpu.VMEM)] * 5,
        out_specs=pl.BlockSpec(memory_space=pltpu.VMEM),
        scratch_shapes=[
            pltpu.VMEM((N_DEV, R, 128), jnp.float32),
            pltpu.SemaphoreType.DMA((4,)),
            pltpu.SemaphoreType.DMA((4,)),
        ],
        compiler_params=pltpu.CompilerParams(collective_id=0),
    )(Q2, K2, V2, bt, lens2)


# device time: 15914 ns/iter; 3.4713x vs baseline; 1.8717x over previous
import os

import jax
import jax.numpy as jnp
from jax import lax
from jax.experimental import pallas as pl
from jax.experimental.pallas import tpu as pltpu

N_DEV = 16
B, H, D, BS, NPAGES_LOCAL = 8, 8, 64, 16, 64
T_LOCAL = NPAGES_LOCAL * BS
HD = H * D
R = H * B
NEG = -(2.0 ** 100)

_SKIP_COMM = os.environ.get("SKIP_COMM") == "1"
_BARRIER_ONLY = os.environ.get("BARRIER_ONLY") == "1"


def kernel(Q, K, V, bt, lens):
    K2 = K.reshape(T_LOCAL, HD)
    V2 = V.reshape(T_LOCAL, HD)
    lens2 = lens.reshape(1, B)

    def body(q_ref, k_ref, v_ref, bt_ref, lens_ref, out_ref,
             gather_ref, send_sems, recv_sems, k_vmem, v_vmem, kv_sems):
        my_pos = lax.axis_index("i")

        k_copy = pltpu.make_async_copy(k_ref, k_vmem, kv_sems.at[0])
        v_copy = pltpu.make_async_copy(v_ref, v_vmem, kv_sems.at[1])
        k_copy.start()
        v_copy.start()

        if not _SKIP_COMM:
            barrier_sem = pltpu.get_barrier_semaphore()
            for d in range(1, N_DEV):
                tgt = lax.rem(my_pos + d, N_DEV)
                pl.semaphore_signal(barrier_sem, inc=1, device_id=(tgt,),
                                    device_id_type=pl.DeviceIdType.MESH)

        lens_col = jnp.transpose(lens_ref[:, :], (1, 0))
        jj = lax.broadcasted_iota(jnp.int32, (B, 64), 1)
        btm = jnp.where(jj < lens_col, bt_ref[:, :], -1)
        gp3 = (lax.broadcasted_iota(jnp.int32, (1, 1, NPAGES_LOCAL), 2)
               + my_pos * NPAGES_LOCAL)
        hitp = (btm.reshape(B, 64, 1) == gp3)
        cntp = jnp.sum(hitp.astype(jnp.float32), axis=1)
        cntp_r = jnp.broadcast_to(cntp[None], (H, B, NPAGES_LOCAL)
                                  ).reshape(R, NPAGES_LOCAL)
        rep = (lax.broadcasted_iota(jnp.int32, (NPAGES_LOCAL, T_LOCAL), 0)
               == lax.broadcasted_iota(jnp.int32, (NPAGES_LOCAL, T_LOCAL), 1)
               // BS).astype(jnp.float32)
        cnt_r = lax.dot_general(cntp_r, rep, (((1,), (0,)), ((), ())),
                                preferred_element_type=jnp.float32)

        q3 = q_ref[:, 0, :, :]
        qall = jnp.transpose(q3, (1, 0, 2)).reshape(R, D)
        ch = lax.broadcasted_iota(jnp.int32, (R, HD), 1) // D
        rh = lax.broadcasted_iota(jnp.int32, (R, HD), 0) // B
        qblk = jnp.where(rh == ch,
                         jnp.broadcast_to(qall[:, None, :], (R, H, D)
                                          ).reshape(R, HD),
                         0.0)

        k_copy.wait()
        k_b = k_vmem[:, :]
        s = lax.dot_general(qblk, k_b, (((1,), (1,)), ((), ())),
                            preferred_element_type=jnp.float32) * (D ** -0.5)
        s = jnp.where(cnt_r > 0.0, s, NEG)
        m_q = jnp.max(s, axis=1, keepdims=True).astype(jnp.bfloat16)
        p = cnt_r * jnp.exp(s - m_q.astype(jnp.float32))
        l = jnp.sum(p, axis=1, keepdims=True)

        v_copy.wait()
        v_b = v_vmem[:, :]
        full = lax.dot_general(p, v_b, (((1,), (0,)), ((), ())),
                               preferred_element_type=jnp.float32)
        masked = jnp.where(rh == ch, full, 0.0)
        fold = (lax.broadcasted_iota(jnp.int32, (HD, D), 0) % D
                == lax.broadcasted_iota(jnp.int32, (HD, D), 1)
                ).astype(jnp.float32)
        acc = lax.dot_general(masked, fold, (((1,), (0,)), ((), ())),
                              preferred_element_type=jnp.float32)

        gather_ref[my_pos, :, 0:D] = acc.astype(jnp.bfloat16)
        gather_ref[my_pos, :, D:D + 1] = m_q
        gather_ref[my_pos, :, D + 1:D + 2] = l.astype(jnp.bfloat16)

        if _SKIP_COMM:
            _merge(gather_ref, out_ref)
            return

        pl.semaphore_wait(barrier_sem, N_DEV - 1)

        if _BARRIER_ONLY:
            _merge(gather_ref, out_ref)
            return

        sends = []
        for d in range(1, N_DEV):
            tgt = lax.rem(my_pos + d, N_DEV)
            rdma = pltpu.make_async_remote_copy(
                src_ref=gather_ref.at[my_pos],
                dst_ref=gather_ref.at[my_pos],
                send_sem=send_sems.at[tgt],
                recv_sem=recv_sems.at[my_pos],
                device_id=(tgt,),
                device_id_type=pl.DeviceIdType.MESH,
            )
            rdma.start()
            sends.append(rdma)
        for d in range(1, N_DEV):
            src = lax.rem(my_pos + N_DEV - d, N_DEV)
            recv = pltpu.make_async_remote_copy(
                src_ref=gather_ref.at[my_pos],
                dst_ref=gather_ref.at[src],
                send_sem=send_sems.at[my_pos],
                recv_sem=recv_sems.at[src],
                device_id=(src,),
                device_id_type=pl.DeviceIdType.MESH,
            )
            recv.wait_recv()
        for rdma in sends:
            rdma.wait_send()

        _merge(gather_ref, out_ref)

    def _merge(gather_ref, out_ref):
        all_blk = gather_ref[:, :, :].astype(jnp.float32)
        acc_a = all_blk[:, :, 0:D]
        m_a = all_blk[:, :, D:D + 1]
        l_a = all_blk[:, :, D + 1:D + 2]
        m_star = jnp.max(m_a, axis=0)
        w = jnp.exp(m_a - m_star[None])
        l_tot = jnp.sum(l_a * w, axis=0)
        out2d = jnp.sum(acc_a * w, axis=0) / l_tot
        for h in range(H):
            out_ref[:, 0, h, :] = out2d[h * B:(h + 1) * B, :]

    return pl.pallas_call(
        body,
        out_shape=jax.ShapeDtypeStruct((B, 1, H, D), jnp.float32),
        in_specs=[
            pl.BlockSpec(memory_space=pltpu.MemorySpace.VMEM),
            pl.BlockSpec(memory_space=pltpu.MemorySpace.HBM),
            pl.BlockSpec(memory_space=pltpu.MemorySpace.HBM),
            pl.BlockSpec(memory_space=pltpu.MemorySpace.VMEM),
            pl.BlockSpec(memory_space=pltpu.MemorySpace.VMEM),
        ],
        out_specs=pl.BlockSpec(memory_space=pltpu.MemorySpace.VMEM),
        scratch_shapes=[
            pltpu.VMEM((N_DEV, R, 128), jnp.bfloat16),
            pltpu.SemaphoreType.DMA((N_DEV,)),
            pltpu.SemaphoreType.DMA((N_DEV,)),
            pltpu.VMEM((T_LOCAL, HD), jnp.float32),
            pltpu.VMEM((T_LOCAL, HD), jnp.float32),
            pltpu.SemaphoreType.DMA((2,)),
        ],
        compiler_params=(None if _SKIP_COMM
                         else pltpu.CompilerParams(collective_id=0)),
    )(Q, K2, V2, bt, lens2)
